# Initial kernel scaffold; baseline (speedup 1.0000x reference)
import jax
import jax.numpy as jnp
from jax.experimental import pallas as pl


def kernel(x, edge_index, Wq, bq, Wk, bk, Wv, bv):
    raise NotImplementedError("write your pallas kernel here")


# speedup vs baseline: 12.9699x; 12.9699x over previous
import dataclasses
import functools

import jax
import jax.numpy as jnp
from jax import lax
from jax.experimental import pallas as pl
from jax.experimental.pallas import tpu as pltpu
from jax.experimental.pallas import tpu_sc as plsc

D = 128
H = 8
DH = D // H

NC = 2
NS = 16
NW = NC * NS
CHUNK = 64
ACC_W = D + 16


def _qkv_body(x_ref, w_ref, b_ref, q_ref, kv_ref):
    acc = jnp.dot(x_ref[...], w_ref[...], preferred_element_type=jnp.float32)
    acc = acc + b_ref[...]
    q_ref[...] = acc[:, :D]
    kv_ref[...] = acc[:, D:]


@functools.lru_cache(maxsize=None)
def _qkv_call(n, blk):
    grid = n // blk
    return pl.pallas_call(
        _qkv_body,
        grid=(grid,),
        in_specs=[
            pl.BlockSpec((blk, D), lambda i: (i, 0)),
            pl.BlockSpec((D, 3 * D), lambda i: (0, 0)),
            pl.BlockSpec((1, 3 * D), lambda i: (0, 0)),
        ],
        out_specs=[
            pl.BlockSpec((blk, D), lambda i: (i, 0)),
            pl.BlockSpec((blk, 2 * D), lambda i: (i, 0)),
        ],
        out_shape=[
            jax.ShapeDtypeStruct((n, D), jnp.float32),
            jax.ShapeDtypeStruct((n, 2 * D), jnp.float32),
        ],
    )


@functools.lru_cache(maxsize=None)
def _sc_edge_call(chunks_per_tile, acc_n):
    rows_per_tile = acc_n // NS
    inv_sqrt_dh = 1.0 / float(DH) ** 0.5

    def body(q_hbm, kv_hbm, src_hbm, dst_hbm, out_hbm,
             src_idx, dst_idx, kvrows, qrows, msg, acc, sem):
        c = lax.axis_index("c")
        s = lax.axis_index("s")
        wid = c * NS + s
        iota = lax.iota(jnp.int32, 16)
        zero16 = jnp.zeros((16,), jnp.float32)

        @pl.loop(0, CHUNK)
        def _(e):
            @pl.loop(0, ACC_W, step=16)
            def _(j):
                msg[e, pl.ds(j, 16)] = zero16

        @pl.loop(0, rows_per_tile, step=CHUNK)
        def _(r):
            pltpu.sync_copy(msg, acc.at[pl.ds(s * rows_per_tile + r, CHUNK)])

        plsc.subcore_barrier()

        base0 = wid * (chunks_per_tile * CHUNK)

        @pl.loop(0, chunks_per_tile)
        def _(ci):
            base = base0 + ci * CHUNK
            pltpu.sync_copy(src_hbm.at[pl.ds(base, CHUNK)], src_idx)
            pltpu.sync_copy(dst_hbm.at[pl.ds(base, CHUNK)], dst_idx)
            ck = pltpu.async_copy(kv_hbm.at[src_idx], kvrows, sem)
            cq = pltpu.async_copy(q_hbm.at[dst_idx], qrows, sem)
            ck.wait()
            cq.wait()

            @pl.loop(0, CHUNK)
            def _(e):
                zvec = zero16
                for h in range(H):
                    kh = kvrows[e, pl.ds(h * DH, DH)]
                    qh = qrows[e, pl.ds(h * DH, DH)]
                    sh = jnp.sum(kh * qh) * inv_sqrt_dh
                    sh = jnp.minimum(jnp.maximum(sh, -5.0), 5.0)
                    ev = jnp.exp(jnp.broadcast_to(sh, (16,)))
                    vh = kvrows[e, pl.ds(D + h * DH, DH)]
                    msg[e, pl.ds(h * DH, DH)] = vh * ev
                    zvec = jnp.where(iota == h, ev, zvec)
                msg[e, pl.ds(D, 16)] = zvec

            pltpu.sync_copy(msg, acc.at[dst_idx], add=True)

        plsc.subcore_barrier()

        @pl.loop(0, rows_per_tile, step=CHUNK)
        def _(r):
            row0 = s * rows_per_tile + r
            pltpu.sync_copy(acc.at[pl.ds(row0, CHUNK)],
                            out_hbm.at[c, pl.ds(row0, CHUNK)])

    cp = pltpu.CompilerParams()
    for f, v in (("needs_layout_passes", False),
                 ("use_tc_tiling_on_sc", False)):
        if f in pltpu.CompilerParams.__dataclass_fields__:
            cp = dataclasses.replace(cp, **{f: v})

    return pl.kernel(
        body,
        out_type=jax.ShapeDtypeStruct((NC, acc_n, ACC_W), jnp.float32),
        mesh=plsc.VectorSubcoreMesh(core_axis_name="c", subcore_axis_name="s"),
        compiler_params=cp,
        scratch_types=[
            pltpu.VMEM((CHUNK,), jnp.int32),
            pltpu.VMEM((CHUNK,), jnp.int32),
            pltpu.VMEM((CHUNK, 2 * D), jnp.float32),
            pltpu.VMEM((CHUNK, D), jnp.float32),
            pltpu.VMEM((CHUNK, ACC_W), jnp.float32),
            pltpu.VMEM_SHARED((acc_n, ACC_W), jnp.float32),
            pltpu.SemaphoreType.DMA,
        ],
    )


def _combine_body(x_ref, p0_ref, p1_ref, o_ref):
    x = x_ref[...]
    wv = p0_ref[0, :, :D] + p1_ref[0, :, :D]
    z = p0_ref[0, :, D:D + H] + p1_ref[0, :, D:D + H]
    r = 1.0 / z
    for h in range(H):
        sl = slice(h * DH, (h + 1) * DH)
        o_ref[:, sl] = x[:, sl] + wv[:, sl] * r[:, h:h + 1]


@functools.lru_cache(maxsize=None)
def _combine_call(n, blk, acc_n):
    grid = n // blk
    return pl.pallas_call(
        _combine_body,
        grid=(grid,),
        in_specs=[
            pl.BlockSpec((blk, D), lambda i: (i, 0)),
            pl.BlockSpec((1, blk, ACC_W), lambda i: (0, i, 0)),
            pl.BlockSpec((1, blk, ACC_W), lambda i: (1, i, 0)),
        ],
        out_specs=pl.BlockSpec((blk, D), lambda i: (i, 0)),
        out_shape=jax.ShapeDtypeStruct((n, D), jnp.float32),
    )


def kernel(x, edge_index, Wq, bq, Wk, bk, Wv, bv):
    n = x.shape[0]
    e = edge_index.shape[1]

    w_cat = jnp.concatenate([Wq, Wk, Wv], axis=1)
    b_cat = jnp.concatenate([bq, bk, bv]).reshape(1, 3 * D)
    q, kv = _qkv_call(n, 2000)(x, w_cat, b_cat)

    chunks_per_tile = -(-e // (NW * CHUNK))
    e_pad = chunks_per_tile * CHUNK * NW
    src = jnp.concatenate(
        [edge_index[0], jnp.zeros((e_pad - e,), jnp.int32)])
    dst = jnp.concatenate(
        [edge_index[1], jnp.full((e_pad - e,), n, jnp.int32)])

    acc_n = -(-(n + 1) // (NS * CHUNK)) * (NS * CHUNK)
    partials = _sc_edge_call(chunks_per_tile, acc_n)(q, kv, src, dst)

    return _combine_call(n, 2000, acc_n)(x, partials, partials)


# speedup vs baseline: 14.7716x; 1.1389x over previous
import dataclasses
import functools

import jax
import jax.numpy as jnp
from jax import lax
from jax.experimental import pallas as pl
from jax.experimental.pallas import tpu as pltpu
from jax.experimental.pallas import tpu_sc as plsc

D = 128
H = 8
DH = D // H

NC = 2
NS = 16
NW = NC * NS
C = 32
SG = 8
ACC_W = D + 16


def _qkv_body(x_ref, w_ref, b_ref, q_ref, kv_ref):
    acc = jnp.dot(x_ref[...], w_ref[...], preferred_element_type=jnp.float32)
    acc = acc + b_ref[...]
    q_ref[...] = acc[:, :D]
    kv_ref[...] = acc[:, D:]


@functools.lru_cache(maxsize=None)
def _qkv_call(n, blk):
    grid = n // blk
    return pl.pallas_call(
        _qkv_body,
        grid=(grid,),
        in_specs=[
            pl.BlockSpec((blk, D), lambda i: (i, 0)),
            pl.BlockSpec((D, 3 * D), lambda i: (0, 0)),
            pl.BlockSpec((1, 3 * D), lambda i: (0, 0)),
        ],
        out_specs=[
            pl.BlockSpec((blk, D), lambda i: (i, 0)),
            pl.BlockSpec((blk, 2 * D), lambda i: (i, 0)),
        ],
        out_shape=[
            jax.ShapeDtypeStruct((n, D), jnp.float32),
            jax.ShapeDtypeStruct((n, 2 * D), jnp.float32),
        ],
    )


@functools.lru_cache(maxsize=None)
def _sc_edge_call(nch, acc_n):
    rows_per_tile = acc_n // NS
    inv_sqrt_dh = 1.0 / float(DH) ** 0.5
    nblocks = nch // SG

    def body(q_hbm, kv_hbm, e2d_hbm, out_hbm,
             kv_a, kv_b, q_a, q_b, msg_a, msg_b, sb, dsts_a, dsts_b, acc,
             sem_sb, sem_ga, sem_gb, sem_sa, sem_sb2):
        cc = lax.axis_index("c")
        ss = lax.axis_index("s")
        wid = cc * NS + ss
        row0 = wid * nch
        iota = lax.iota(jnp.int32, 16)
        zero16 = jnp.zeros((16,), jnp.float32)
        masks = [iota == h for h in range(H)]

        @pl.loop(0, C)
        def _(e):
            @pl.loop(0, ACC_W, step=16)
            def _(j):
                msg_a[e, pl.ds(j, 16)] = zero16

        @pl.loop(0, rows_per_tile, step=C)
        def _(r):
            pltpu.sync_copy(msg_a, acc.at[pl.ds(ss * rows_per_tile + r, C)])

        plsc.subcore_barrier()

        def sb_fetch(b, half):
            return pltpu.make_async_copy(
                e2d_hbm.at[pl.ds(row0 + b * SG, SG)], sb.at[half], sem_sb)

        def gathers(ci, kv_t, q_t, sem_t):
            h = (ci // SG) % 2
            srow = ci % SG
            gk = pltpu.make_async_copy(
                kv_hbm.at[sb.at[h, srow, 0]], kv_t, sem_t)
            gq = pltpu.make_async_copy(
                q_hbm.at[sb.at[h, srow, 1]], q_t, sem_t)
            return gk, gq

        def scatter(msg_t, dsts_t, sem_t):
            return pltpu.make_async_copy(msg_t, acc.at[dsts_t], sem_t)

        def compute(kv_t, q_t, msg_t):
            @pl.loop(0, C, step=2)
            def _(e0):
                for e in (e0, e0 + 1):
                    zvec = zero16
                    for h in range(H):
                        kh = kv_t[e, pl.ds(h * DH, DH)]
                        qh = q_t[e, pl.ds(h * DH, DH)]
                        sh = jnp.sum(kh * qh) * inv_sqrt_dh
                        sh = jnp.minimum(jnp.maximum(sh, -5.0), 5.0)
                        ev = jnp.exp(jnp.broadcast_to(sh, (16,)))
                        vh = kv_t[e, pl.ds(D + h * DH, DH)]
                        msg_t[e, pl.ds(h * DH, DH)] = vh * ev
                        zvec = jnp.where(masks[h], ev, zvec)
                    msg_t[e, pl.ds(D, 16)] = zvec

        def phase(ci, kv_t, q_t, msg_t, dsts_t, sem_gt, sem_st,
                  kv_n, q_n, sem_gn):
            nxt = ci + 1
            h = (ci // SG) % 2
            srow = ci % SG

            @pl.when(jnp.logical_and(nxt % SG == 0, nxt < nch))
            def _():
                sb_fetch(nxt // SG, (nxt // SG) % 2).wait()

            @pl.when(nxt < nch)
            def _():
                gk, gq = gathers(nxt, kv_n, q_n, sem_gn)
                gk.start()
                gq.start()

            gk, gq = gathers(ci, kv_t, q_t, sem_gt)
            gk.wait()
            gq.wait()

            @pl.when(ci >= 2)
            def _():
                scatter(msg_t, dsts_t, sem_st).wait()

            compute(kv_t, q_t, msg_t)

            for j in range(0, C, 16):
                dsts_t[pl.ds(j, 16)] = sb[h, srow, 1, pl.ds(j, 16)]
            scatter(msg_t, dsts_t, sem_st).start(add=True)

            @pl.when(jnp.logical_and(nxt % SG == 0,
                                     nxt // SG + 1 < nblocks))
            def _():
                bb = nxt // SG + 1
                sb_fetch(bb, bb % 2).start()

        sb_fetch(0, 0).start()
        sb_fetch(1, 1).start()
        sb_fetch(0, 0).wait()
        gk, gq = gathers(0, kv_a, q_a, sem_ga)
        gk.start()
        gq.start()

        @pl.loop(0, nch, step=2)
        def _(ci):
            phase(ci, kv_a, q_a, msg_a, dsts_a, sem_ga, sem_sa,
                  kv_b, q_b, sem_gb)
            phase(ci + 1, kv_b, q_b, msg_b, dsts_b, sem_gb, sem_sb2,
                  kv_a, q_a, sem_ga)

        scatter(msg_a, dsts_a, sem_sa).wait()
        scatter(msg_b, dsts_b, sem_sb2).wait()

        plsc.subcore_barrier()

        @pl.loop(0, rows_per_tile, step=C)
        def _(r):
            rr = ss * rows_per_tile + r
            pltpu.sync_copy(acc.at[pl.ds(rr, C)],
                            out_hbm.at[cc, pl.ds(rr, C)])

    cp = pltpu.CompilerParams()
    for f, v in (("needs_layout_passes", False),
                 ("use_tc_tiling_on_sc", False)):
        if f in pltpu.CompilerParams.__dataclass_fields__:
            cp = dataclasses.replace(cp, **{f: v})

    return pl.kernel(
        body,
        out_type=jax.ShapeDtypeStruct((NC, acc_n, ACC_W), jnp.float32),
        mesh=plsc.VectorSubcoreMesh(core_axis_name="c", subcore_axis_name="s"),
        compiler_params=cp,
        scratch_types=[
            pltpu.VMEM((C, 2 * D), jnp.float32),
            pltpu.VMEM((C, 2 * D), jnp.float32),
            pltpu.VMEM((C, D), jnp.float32),
            pltpu.VMEM((C, D), jnp.float32),
            pltpu.VMEM((C, ACC_W), jnp.float32),
            pltpu.VMEM((C, ACC_W), jnp.float32),
            pltpu.VMEM((2, SG, 2, C), jnp.int32),
            pltpu.VMEM((C,), jnp.int32),
            pltpu.VMEM((C,), jnp.int32),
            pltpu.VMEM_SHARED((acc_n, ACC_W), jnp.float32),
            pltpu.SemaphoreType.DMA,
            pltpu.SemaphoreType.DMA,
            pltpu.SemaphoreType.DMA,
            pltpu.SemaphoreType.DMA,
            pltpu.SemaphoreType.DMA,
        ],
    )


def _combine_body(x_ref, p0_ref, p1_ref, o_ref):
    x = x_ref[...]
    wv = p0_ref[0, :, :D] + p1_ref[0, :, :D]
    z = p0_ref[0, :, D:D + H] + p1_ref[0, :, D:D + H]
    r = 1.0 / z
    for h in range(H):
        sl = slice(h * DH, (h + 1) * DH)
        o_ref[:, sl] = x[:, sl] + wv[:, sl] * r[:, h:h + 1]


@functools.lru_cache(maxsize=None)
def _combine_call(n, blk, acc_n):
    grid = n // blk
    return pl.pallas_call(
        _combine_body,
        grid=(grid,),
        in_specs=[
            pl.BlockSpec((blk, D), lambda i: (i, 0)),
            pl.BlockSpec((1, blk, ACC_W), lambda i: (0, i, 0)),
            pl.BlockSpec((1, blk, ACC_W), lambda i: (1, i, 0)),
        ],
        out_specs=pl.BlockSpec((blk, D), lambda i: (i, 0)),
        out_shape=jax.ShapeDtypeStruct((n, D), jnp.float32),
    )


def kernel(x, edge_index, Wq, bq, Wk, bk, Wv, bv):
    n = x.shape[0]
    e = edge_index.shape[1]

    w_cat = jnp.concatenate([Wq, Wk, Wv], axis=1)
    b_cat = jnp.concatenate([bq, bk, bv]).reshape(1, 3 * D)
    q, kv = _qkv_call(n, 2000)(x, w_cat, b_cat)

    nch = -(-e // (NW * C))
    nch = -(-nch // SG) * SG
    e_pad = nch * C * NW
    npad = e_pad - e
    acc_n = -(-(n + 1) // (NS * C)) * (NS * C)
    src = jnp.concatenate(
        [edge_index[0], jnp.zeros((npad,), jnp.int32)])
    dst = jnp.concatenate(
        [edge_index[1], n + (jnp.arange(npad, dtype=jnp.int32)
                             % (acc_n - n))])
    e2d = jnp.stack([src.reshape(-1, C), dst.reshape(-1, C)], axis=1)

    partials = _sc_edge_call(nch, acc_n)(q, kv, e2d)

    return _combine_call(n, 2000, acc_n)(x, partials, partials)


# speedup vs baseline: 54.0809x; 3.6611x over previous
import dataclasses
import functools

import jax
import jax.numpy as jnp
from jax import lax
from jax.experimental import pallas as pl
from jax.experimental.pallas import tpu as pltpu
from jax.experimental.pallas import tpu_sc as plsc

D = 128
H = 8
DH = D // H

NC = 2
NS = 16
NW = NC * NS
C = 32
SG = 8
ACC_W = D + 16


def _qkv_body(x_ref, w_ref, b_ref, q_ref, kv_ref):
    acc = jnp.dot(x_ref[...], w_ref[...], preferred_element_type=jnp.float32)
    acc = acc + b_ref[...]
    q_ref[...] = acc[:, :D] * (1.0 / float(DH) ** 0.5)
    kv_ref[...] = acc[:, D:]


@functools.lru_cache(maxsize=None)
def _qkv_call(n, blk):
    grid = n // blk
    return pl.pallas_call(
        _qkv_body,
        grid=(grid,),
        in_specs=[
            pl.BlockSpec((blk, D), lambda i: (i, 0)),
            pl.BlockSpec((D, 3 * D), lambda i: (0, 0)),
            pl.BlockSpec((1, 3 * D), lambda i: (0, 0)),
        ],
        out_specs=[
            pl.BlockSpec((blk, D), lambda i: (i, 0)),
            pl.BlockSpec((blk, 2 * D), lambda i: (i, 0)),
        ],
        out_shape=[
            jax.ShapeDtypeStruct((n, D), jnp.float32),
            jax.ShapeDtypeStruct((n, 2 * D), jnp.float32),
        ],
    )


@functools.lru_cache(maxsize=None)
def _sc_edge_call(nch, acc_n):
    rows_per_tile = acc_n // NS
    inv_sqrt_dh = 1.0 / float(DH) ** 0.5
    nblocks = nch // SG

    def body(q_hbm, kv_hbm, e2d_hbm, out_hbm,
             kv_a, kv_b, q_a, q_b, msg_a, msg_b, sb, dsts_a, dsts_b, acc,
             sem_sb, sem_ga, sem_gb, sem_sa, sem_sb2):
        cc = lax.axis_index("c")
        ss = lax.axis_index("s")
        wid = cc * NS + ss
        row0 = wid * nch
        iota = lax.iota(jnp.int32, 16)
        zero16 = jnp.zeros((16,), jnp.float32)
        masks = [iota == h for h in range(H)]

        @pl.loop(0, C)
        def _(e):
            @pl.loop(0, ACC_W, step=16)
            def _(j):
                msg_a[e, pl.ds(j, 16)] = zero16

        @pl.loop(0, rows_per_tile, step=C)
        def _(r):
            pltpu.sync_copy(msg_a, acc.at[pl.ds(ss * rows_per_tile + r, C)])

        plsc.subcore_barrier()

        def sb_fetch(b, half):
            return pltpu.make_async_copy(
                e2d_hbm.at[pl.ds(row0 + b * SG, SG)], sb.at[half], sem_sb)

        def gathers(ci, kv_t, q_t, sem_t):
            h = (ci // SG) % 2
            srow = ci % SG
            gk = pltpu.make_async_copy(
                kv_hbm.at[sb.at[h, srow, 0]], kv_t, sem_t)
            gq = pltpu.make_async_copy(
                q_hbm.at[sb.at[h, srow, 1]], q_t, sem_t)
            return gk, gq

        def scatter(msg_t, dsts_t, sem_t):
            return pltpu.make_async_copy(msg_t, acc.at[dsts_t], sem_t)

        lane15 = jnp.full((16, 1), 15, jnp.int32)
        gd = lax.GatherDimensionNumbers(
            offset_dims=(), collapsed_slice_dims=(0,), start_index_map=(0,))

        def bcast_last(ps):
            return lax.gather(ps, lane15, gd, slice_sizes=(1,),
                              mode=lax.GatherScatterMode.PROMISE_IN_BOUNDS)

        def compute(kv_t, q_t, msg_t):
            @plsc.parallel_loop(0, C, step=1, unroll=4)
            def _(e):
                zvec = zero16
                for h in range(H):
                    kh = kv_t[e, pl.ds(h * DH, DH)]
                    qh = q_t[e, pl.ds(h * DH, DH)]
                    ps = jnp.cumsum(kh * qh)
                    sv = bcast_last(ps)
                    sv = jnp.minimum(jnp.maximum(sv, -5.0), 5.0)
                    ev = jnp.exp(sv)
                    vh = kv_t[e, pl.ds(D + h * DH, DH)]
                    msg_t[e, pl.ds(h * DH, DH)] = vh * ev
                    zvec = jnp.where(masks[h], ev, zvec)
                msg_t[e, pl.ds(D, 16)] = zvec

        def phase(ci, kv_t, q_t, msg_t, dsts_t, sem_gt, sem_st,
                  kv_n, q_n, sem_gn):
            nxt = ci + 1
            h = (ci // SG) % 2
            srow = ci % SG

            @pl.when(jnp.logical_and(nxt % SG == 0, nxt < nch))
            def _():
                sb_fetch(nxt // SG, (nxt // SG) % 2).wait()

            @pl.when(nxt < nch)
            def _():
                gk, gq = gathers(nxt, kv_n, q_n, sem_gn)
                gk.start()
                gq.start()

            gk, gq = gathers(ci, kv_t, q_t, sem_gt)
            gk.wait()
            gq.wait()

            @pl.when(ci >= 2)
            def _():
                scatter(msg_t, dsts_t, sem_st).wait()

            compute(kv_t, q_t, msg_t)

            for j in range(0, C, 16):
                dsts_t[pl.ds(j, 16)] = sb[h, srow, 1, pl.ds(j, 16)]
            scatter(msg_t, dsts_t, sem_st).start(add=True)

            @pl.when(jnp.logical_and(nxt % SG == 0,
                                     nxt // SG + 1 < nblocks))
            def _():
                bb = nxt // SG + 1
                sb_fetch(bb, bb % 2).start()

        sb_fetch(0, 0).start()
        sb_fetch(1, 1).start()
        sb_fetch(0, 0).wait()
        gk, gq = gathers(0, kv_a, q_a, sem_ga)
        gk.start()
        gq.start()

        @pl.loop(0, nch, step=2)
        def _(ci):
            phase(ci, kv_a, q_a, msg_a, dsts_a, sem_ga, sem_sa,
                  kv_b, q_b, sem_gb)
            phase(ci + 1, kv_b, q_b, msg_b, dsts_b, sem_gb, sem_sb2,
                  kv_a, q_a, sem_ga)

        scatter(msg_a, dsts_a, sem_sa).wait()
        scatter(msg_b, dsts_b, sem_sb2).wait()

        plsc.subcore_barrier()

        @pl.loop(0, rows_per_tile, step=C)
        def _(r):
            rr = ss * rows_per_tile + r
            pltpu.sync_copy(acc.at[pl.ds(rr, C)],
                            out_hbm.at[cc, pl.ds(rr, C)])

    cp = pltpu.CompilerParams()
    for f, v in (("needs_layout_passes", False),
                 ("use_tc_tiling_on_sc", False)):
        if f in pltpu.CompilerParams.__dataclass_fields__:
            cp = dataclasses.replace(cp, **{f: v})

    return pl.kernel(
        body,
        out_type=jax.ShapeDtypeStruct((NC, acc_n, ACC_W), jnp.float32),
        mesh=plsc.VectorSubcoreMesh(core_axis_name="c", subcore_axis_name="s"),
        compiler_params=cp,
        scratch_types=[
            pltpu.VMEM((C, 2 * D), jnp.float32),
            pltpu.VMEM((C, 2 * D), jnp.float32),
            pltpu.VMEM((C, D), jnp.float32),
            pltpu.VMEM((C, D), jnp.float32),
            pltpu.VMEM((C, ACC_W), jnp.float32),
            pltpu.VMEM((C, ACC_W), jnp.float32),
            pltpu.VMEM((2, SG, 2, C), jnp.int32),
            pltpu.VMEM((C,), jnp.int32),
            pltpu.VMEM((C,), jnp.int32),
            pltpu.VMEM_SHARED((acc_n, ACC_W), jnp.float32),
            pltpu.SemaphoreType.DMA,
            pltpu.SemaphoreType.DMA,
            pltpu.SemaphoreType.DMA,
            pltpu.SemaphoreType.DMA,
            pltpu.SemaphoreType.DMA,
        ],
    )


def _combine_body(x_ref, p0_ref, p1_ref, o_ref):
    x = x_ref[...]
    wv = p0_ref[0, :, :D] + p1_ref[0, :, :D]
    z = p0_ref[0, :, D:D + H] + p1_ref[0, :, D:D + H]
    r = 1.0 / z
    for h in range(H):
        sl = slice(h * DH, (h + 1) * DH)
        o_ref[:, sl] = x[:, sl] + wv[:, sl] * r[:, h:h + 1]


@functools.lru_cache(maxsize=None)
def _combine_call(n, blk, acc_n):
    grid = n // blk
    return pl.pallas_call(
        _combine_body,
        grid=(grid,),
        in_specs=[
            pl.BlockSpec((blk, D), lambda i: (i, 0)),
            pl.BlockSpec((1, blk, ACC_W), lambda i: (0, i, 0)),
            pl.BlockSpec((1, blk, ACC_W), lambda i: (1, i, 0)),
        ],
        out_specs=pl.BlockSpec((blk, D), lambda i: (i, 0)),
        out_shape=jax.ShapeDtypeStruct((n, D), jnp.float32),
    )


def kernel(x, edge_index, Wq, bq, Wk, bk, Wv, bv):
    n = x.shape[0]
    e = edge_index.shape[1]

    w_cat = jnp.concatenate([Wq, Wk, Wv], axis=1)
    b_cat = jnp.concatenate([bq, bk, bv]).reshape(1, 3 * D)
    q, kv = _qkv_call(n, 2000)(x, w_cat, b_cat)

    nch = -(-e // (NW * C))
    nch = -(-nch // SG) * SG
    e_pad = nch * C * NW
    npad = e_pad - e
    acc_n = -(-(n + 1) // (NS * C)) * (NS * C)
    src = jnp.concatenate(
        [edge_index[0], jnp.zeros((npad,), jnp.int32)])
    dst = jnp.concatenate(
        [edge_index[1], n + (jnp.arange(npad, dtype=jnp.int32)
                             % (acc_n - n))])
    e2d = jnp.stack([src.reshape(-1, C), dst.reshape(-1, C)], axis=1)

    partials = _sc_edge_call(nch, acc_n)(q, kv, e2d)

    return _combine_call(n, 2000, acc_n)(x, partials, partials)


# speedup vs baseline: 54.8845x; 1.0149x over previous
import dataclasses
import functools

import jax
import jax.numpy as jnp
from jax import lax
from jax.experimental import pallas as pl
from jax.experimental.pallas import tpu as pltpu
from jax.experimental.pallas import tpu_sc as plsc

D = 128
H = 8
DH = D // H

NC = 2
NS = 16
NW = NC * NS
C = 32
SG = 16
ACC_W = D + 16


def _qkv_body(x_ref, w_ref, b_ref, q_ref, kv_ref):
    acc = jnp.dot(x_ref[...], w_ref[...], preferred_element_type=jnp.float32)
    acc = acc + b_ref[...]
    q_ref[...] = acc[:, :D] * (1.0 / float(DH) ** 0.5)
    kv_ref[...] = acc[:, D:]


@functools.lru_cache(maxsize=None)
def _qkv_call(n, blk):
    grid = n // blk
    return pl.pallas_call(
        _qkv_body,
        grid=(grid,),
        in_specs=[
            pl.BlockSpec((blk, D), lambda i: (i, 0)),
            pl.BlockSpec((D, 3 * D), lambda i: (0, 0)),
            pl.BlockSpec((1, 3 * D), lambda i: (0, 0)),
        ],
        out_specs=[
            pl.BlockSpec((blk, D), lambda i: (i, 0)),
            pl.BlockSpec((blk, 2 * D), lambda i: (i, 0)),
        ],
        out_shape=[
            jax.ShapeDtypeStruct((n, D), jnp.float32),
            jax.ShapeDtypeStruct((n, 2 * D), jnp.float32),
        ],
    )


@functools.lru_cache(maxsize=None)
def _sc_edge_call(nch, acc_n):
    rows_per_tile = acc_n // NS
    inv_sqrt_dh = 1.0 / float(DH) ** 0.5
    nblocks = nch // SG

    def body(q_hbm, kv_hbm, e2d_hbm, out_hbm,
             kv_a, kv_b, q_a, q_b, msg_a, msg_b, sb, dsts_a, dsts_b, acc,
             sem_sb, sem_ga, sem_gb, sem_sa, sem_sb2):
        cc = lax.axis_index("c")
        ss = lax.axis_index("s")
        wid = cc * NS + ss
        row0 = wid * nch
        iota = lax.iota(jnp.int32, 16)
        zero16 = jnp.zeros((16,), jnp.float32)
        masks = [iota == h for h in range(H)]

        @pl.loop(0, C)
        def _(e):
            @pl.loop(0, ACC_W, step=16)
            def _(j):
                msg_a[e, pl.ds(j, 16)] = zero16

        @pl.loop(0, rows_per_tile, step=C)
        def _(r):
            pltpu.sync_copy(msg_a, acc.at[pl.ds(ss * rows_per_tile + r, C)])

        plsc.subcore_barrier()

        def sb_fetch(b, half):
            return pltpu.make_async_copy(
                e2d_hbm.at[pl.ds(row0 + b * SG, SG)], sb.at[half], sem_sb)

        def gathers(ci, kv_t, q_t, sem_t):
            h = (ci // SG) % 2
            srow = ci % SG
            gk = pltpu.make_async_copy(
                kv_hbm.at[sb.at[h, srow, 0]], kv_t, sem_t)
            gq = pltpu.make_async_copy(
                q_hbm.at[sb.at[h, srow, 1]], q_t, sem_t)
            return gk, gq

        def scatter(msg_t, dsts_t, sem_t):
            return pltpu.make_async_copy(msg_t, acc.at[dsts_t], sem_t)

        lane15 = jnp.full((16, 1), 15, jnp.int32)
        gd = lax.GatherDimensionNumbers(
            offset_dims=(), collapsed_slice_dims=(0,), start_index_map=(0,))

        def bcast_last(ps):
            return lax.gather(ps, lane15, gd, slice_sizes=(1,),
                              mode=lax.GatherScatterMode.PROMISE_IN_BOUNDS)

        def compute(kv_t, q_t, msg_t):
            @plsc.parallel_loop(0, C, step=1, unroll=2)
            def _(e):
                zvec = zero16
                for h in range(H):
                    kh = kv_t[e, pl.ds(h * DH, DH)]
                    qh = q_t[e, pl.ds(h * DH, DH)]
                    ps = jnp.cumsum(kh * qh)
                    sv = bcast_last(ps)
                    sv = jnp.minimum(jnp.maximum(sv, -5.0), 5.0)
                    ev = jnp.exp(sv)
                    vh = kv_t[e, pl.ds(D + h * DH, DH)]
                    msg_t[e, pl.ds(h * DH, DH)] = vh * ev
                    zvec = jnp.where(masks[h], ev, zvec)
                msg_t[e, pl.ds(D, 16)] = zvec

        def phase(ci, kv_t, q_t, msg_t, dsts_t, sem_gt, sem_st,
                  kv_n, q_n, sem_gn):
            nxt = ci + 1
            h = (ci // SG) % 2
            srow = ci % SG

            @pl.when(jnp.logical_and(nxt % SG == 0, nxt < nch))
            def _():
                sb_fetch(nxt // SG, (nxt // SG) % 2).wait()

            @pl.when(nxt < nch)
            def _():
                gk, gq = gathers(nxt, kv_n, q_n, sem_gn)
                gk.start()
                gq.start()

            gk, gq = gathers(ci, kv_t, q_t, sem_gt)
            gk.wait()
            gq.wait()

            @pl.when(ci >= 2)
            def _():
                scatter(msg_t, dsts_t, sem_st).wait()

            compute(kv_t, q_t, msg_t)

            for j in range(0, C, 16):
                dsts_t[pl.ds(j, 16)] = sb[h, srow, 1, pl.ds(j, 16)]
            scatter(msg_t, dsts_t, sem_st).start(add=True)

            @pl.when(jnp.logical_and(nxt % SG == 0,
                                     nxt // SG + 1 < nblocks))
            def _():
                bb = nxt // SG + 1
                sb_fetch(bb, bb % 2).start()

        sb_fetch(0, 0).start()
        sb_fetch(1, 1).start()
        sb_fetch(0, 0).wait()
        gk, gq = gathers(0, kv_a, q_a, sem_ga)
        gk.start()
        gq.start()

        @pl.loop(0, nch, step=2)
        def _(ci):
            phase(ci, kv_a, q_a, msg_a, dsts_a, sem_ga, sem_sa,
                  kv_b, q_b, sem_gb)
            phase(ci + 1, kv_b, q_b, msg_b, dsts_b, sem_gb, sem_sb2,
                  kv_a, q_a, sem_ga)

        scatter(msg_a, dsts_a, sem_sa).wait()
        scatter(msg_b, dsts_b, sem_sb2).wait()

        plsc.subcore_barrier()

        @pl.loop(0, rows_per_tile, step=C)
        def _(r):
            rr = ss * rows_per_tile + r
            pltpu.sync_copy(acc.at[pl.ds(rr, C)],
                            out_hbm.at[cc, pl.ds(rr, C)])

    cp = pltpu.CompilerParams()
    for f, v in (("needs_layout_passes", False),
                 ("use_tc_tiling_on_sc", False)):
        if f in pltpu.CompilerParams.__dataclass_fields__:
            cp = dataclasses.replace(cp, **{f: v})

    return pl.kernel(
        body,
        out_type=jax.ShapeDtypeStruct((NC, acc_n, ACC_W), jnp.float32),
        mesh=plsc.VectorSubcoreMesh(core_axis_name="c", subcore_axis_name="s"),
        compiler_params=cp,
        scratch_types=[
            pltpu.VMEM((C, 2 * D), jnp.float32),
            pltpu.VMEM((C, 2 * D), jnp.float32),
            pltpu.VMEM((C, D), jnp.float32),
            pltpu.VMEM((C, D), jnp.float32),
            pltpu.VMEM((C, ACC_W), jnp.float32),
            pltpu.VMEM((C, ACC_W), jnp.float32),
            pltpu.VMEM((2, SG, 2, C), jnp.int32),
            pltpu.VMEM((C,), jnp.int32),
            pltpu.VMEM((C,), jnp.int32),
            pltpu.VMEM_SHARED((acc_n, ACC_W), jnp.float32),
            pltpu.SemaphoreType.DMA,
            pltpu.SemaphoreType.DMA,
            pltpu.SemaphoreType.DMA,
            pltpu.SemaphoreType.DMA,
            pltpu.SemaphoreType.DMA,
        ],
    )


def _combine_body(x_ref, p0_ref, p1_ref, o_ref):
    x = x_ref[...]
    wv = p0_ref[0, :, :D] + p1_ref[0, :, :D]
    z = p0_ref[0, :, D:D + H] + p1_ref[0, :, D:D + H]
    r = 1.0 / z
    for h in range(H):
        sl = slice(h * DH, (h + 1) * DH)
        o_ref[:, sl] = x[:, sl] + wv[:, sl] * r[:, h:h + 1]


@functools.lru_cache(maxsize=None)
def _combine_call(n, blk, acc_n):
    grid = n // blk
    return pl.pallas_call(
        _combine_body,
        grid=(grid,),
        in_specs=[
            pl.BlockSpec((blk, D), lambda i: (i, 0)),
            pl.BlockSpec((1, blk, ACC_W), lambda i: (0, i, 0)),
            pl.BlockSpec((1, blk, ACC_W), lambda i: (1, i, 0)),
        ],
        out_specs=pl.BlockSpec((blk, D), lambda i: (i, 0)),
        out_shape=jax.ShapeDtypeStruct((n, D), jnp.float32),
    )


def kernel(x, edge_index, Wq, bq, Wk, bk, Wv, bv):
    n = x.shape[0]
    e = edge_index.shape[1]

    w_cat = jnp.concatenate([Wq, Wk, Wv], axis=1)
    b_cat = jnp.concatenate([bq, bk, bv]).reshape(1, 3 * D)
    q, kv = _qkv_call(n, 2000)(x, w_cat, b_cat)

    nch = -(-e // (NW * C))
    nch = -(-nch // SG) * SG
    e_pad = nch * C * NW
    npad = e_pad - e
    acc_n = -(-(n + 1) // (NS * C)) * (NS * C)
    src = jnp.concatenate(
        [edge_index[0], jnp.zeros((npad,), jnp.int32)])
    dst = jnp.concatenate(
        [edge_index[1], n + (jnp.arange(npad, dtype=jnp.int32)
                             % (acc_n - n))])
    e2d = jnp.stack([src.reshape(-1, C), dst.reshape(-1, C)], axis=1)

    partials = _sc_edge_call(nch, acc_n)(q, kv, e2d)

    return _combine_call(n, 2000, acc_n)(x, partials, partials)


# speedup vs baseline: 56.9935x; 1.0384x over previous
import dataclasses
import functools

import jax
import jax.numpy as jnp
from jax import lax
from jax.experimental import pallas as pl
from jax.experimental.pallas import tpu as pltpu
from jax.experimental.pallas import tpu_sc as plsc

D = 128
H = 8
DH = D // H

NC = 2
NS = 16
NW = NC * NS
C = 32
SG = 16
ACC_W = D + 16


def _qkv_body(x_ref, w_ref, b_ref, q_ref, kv_ref):
    acc = jnp.dot(x_ref[...], w_ref[...], preferred_element_type=jnp.float32)
    acc = acc + b_ref[...]
    q_ref[...] = acc[:, :D] * (1.0 / float(DH) ** 0.5)
    kv_ref[...] = acc[:, D:]


@functools.lru_cache(maxsize=None)
def _qkv_call(n, blk):
    grid = n // blk
    return pl.pallas_call(
        _qkv_body,
        grid=(grid,),
        in_specs=[
            pl.BlockSpec((blk, D), lambda i: (i, 0)),
            pl.BlockSpec((D, 3 * D), lambda i: (0, 0)),
            pl.BlockSpec((1, 3 * D), lambda i: (0, 0)),
        ],
        out_specs=[
            pl.BlockSpec((blk, D), lambda i: (i, 0)),
            pl.BlockSpec((blk, 2 * D), lambda i: (i, 0)),
        ],
        out_shape=[
            jax.ShapeDtypeStruct((n, D), jnp.float32),
            jax.ShapeDtypeStruct((n, 2 * D), jnp.float32),
        ],
    )


@functools.lru_cache(maxsize=None)
def _sc_edge_call(nch, acc_n):
    rows_per_tile = acc_n // NS
    inv_sqrt_dh = 1.0 / float(DH) ** 0.5
    nblocks = nch // SG

    def body(q_hbm, kv_hbm, e2d_hbm, out_hbm,
             kv_a, kv_b, q_a, q_b, msg_a, msg_b, sb, dsts_a, dsts_b, acc,
             sem_sb, sem_ga, sem_gb, sem_sa, sem_sb2):
        cc = lax.axis_index("c")
        ss = lax.axis_index("s")
        wid = cc * NS + ss
        row0 = wid * nch
        iota = lax.iota(jnp.int32, 16)
        zero16 = jnp.zeros((16,), jnp.float32)
        masks = [iota == h for h in range(H)]

        @pl.loop(0, C)
        def _(e):
            @pl.loop(0, ACC_W, step=16)
            def _(j):
                msg_a[e, pl.ds(j, 16)] = zero16

        @pl.loop(0, rows_per_tile, step=C)
        def _(r):
            pltpu.sync_copy(msg_a, acc.at[pl.ds(ss * rows_per_tile + r, C)])

        plsc.subcore_barrier()

        def sb_fetch(b, half):
            return pltpu.make_async_copy(
                e2d_hbm.at[pl.ds(row0 + b * SG, SG)], sb.at[half], sem_sb)

        def gathers(ci, kv_t, q_t, sem_t):
            h = (ci // SG) % 2
            srow = ci % SG
            gk = pltpu.make_async_copy(
                kv_hbm.at[sb.at[h, srow, 0]], kv_t, sem_t)
            gq = pltpu.make_async_copy(
                q_hbm.at[sb.at[h, srow, 1]], q_t, sem_t)
            return gk, gq

        def scatter(msg_t, dsts_t, sem_t):
            return pltpu.make_async_copy(msg_t, acc.at[dsts_t], sem_t)

        lane15 = jnp.full((16, 1), 15, jnp.int32)
        gd = lax.GatherDimensionNumbers(
            offset_dims=(), collapsed_slice_dims=(0,), start_index_map=(0,))

        def bcast_last(ps):
            return lax.gather(ps, lane15, gd, slice_sizes=(1,),
                              mode=lax.GatherScatterMode.PROMISE_IN_BOUNDS)

        def compute(kv_t, q_t, msg_t):
            @plsc.parallel_loop(0, C, step=1, unroll=2)
            def _(e):
                zvec = zero16
                for h in range(H):
                    kh = kv_t[e, pl.ds(h * DH, DH)]
                    qh = q_t[e, pl.ds(h * DH, DH)]
                    ps = jnp.cumsum(kh * qh)
                    sv = bcast_last(ps)
                    sv = jnp.minimum(jnp.maximum(sv, -5.0), 5.0)
                    ev = jnp.exp(sv)
                    vh = kv_t[e, pl.ds(D + h * DH, DH)]
                    msg_t[e, pl.ds(h * DH, DH)] = vh * ev
                    zvec = jnp.where(masks[h], ev, zvec)
                msg_t[e, pl.ds(D, 16)] = zvec

        def phase(ci, kv_t, q_t, msg_t, dsts_t, sem_gt, sem_st,
                  kv_n, q_n, sem_gn):
            nxt = ci + 1
            h = (ci // SG) % 2
            srow = ci % SG

            @pl.when(jnp.logical_and(nxt % SG == 0, nxt < nch))
            def _():
                sb_fetch(nxt // SG, (nxt // SG) % 2).wait()

            @pl.when(nxt < nch)
            def _():
                gk, gq = gathers(nxt, kv_n, q_n, sem_gn)
                gk.start()
                gq.start()

            gk, gq = gathers(ci, kv_t, q_t, sem_gt)
            gk.wait()
            gq.wait()

            @pl.when(ci >= 2)
            def _():
                scatter(msg_t, dsts_t, sem_st).wait()

            compute(kv_t, q_t, msg_t)

            for j in range(0, C, 16):
                dsts_t[pl.ds(j, 16)] = sb[h, srow, 1, pl.ds(j, 16)]
            scatter(msg_t, dsts_t, sem_st).start(add=True)

            @pl.when(jnp.logical_and(nxt % SG == 0,
                                     nxt // SG + 1 < nblocks))
            def _():
                bb = nxt // SG + 1
                sb_fetch(bb, bb % 2).start()

        sb_fetch(0, 0).start()
        sb_fetch(1, 1).start()
        sb_fetch(0, 0).wait()
        gk, gq = gathers(0, kv_a, q_a, sem_ga)
        gk.start()
        gq.start()

        @pl.loop(0, nch, step=2)
        def _(ci):
            phase(ci, kv_a, q_a, msg_a, dsts_a, sem_ga, sem_sa,
                  kv_b, q_b, sem_gb)
            phase(ci + 1, kv_b, q_b, msg_b, dsts_b, sem_gb, sem_sb2,
                  kv_a, q_a, sem_ga)

        scatter(msg_a, dsts_a, sem_sa).wait()
        scatter(msg_b, dsts_b, sem_sb2).wait()

        plsc.subcore_barrier()

        @pl.loop(0, rows_per_tile, step=C)
        def _(r):
            rr = ss * rows_per_tile + r
            pltpu.sync_copy(acc.at[pl.ds(rr, C)],
                            out_hbm.at[cc, pl.ds(rr, C)])

    cp = pltpu.CompilerParams()
    for f, v in (("needs_layout_passes", False),
                 ("use_tc_tiling_on_sc", False)):
        if f in pltpu.CompilerParams.__dataclass_fields__:
            cp = dataclasses.replace(cp, **{f: v})

    return pl.kernel(
        body,
        out_type=jax.ShapeDtypeStruct((NC, acc_n, ACC_W), jnp.float32),
        mesh=plsc.VectorSubcoreMesh(core_axis_name="c", subcore_axis_name="s"),
        compiler_params=cp,
        scratch_types=[
            pltpu.VMEM((C, 2 * D), jnp.float32),
            pltpu.VMEM((C, 2 * D), jnp.float32),
            pltpu.VMEM((C, D), jnp.float32),
            pltpu.VMEM((C, D), jnp.float32),
            pltpu.VMEM((C, ACC_W), jnp.float32),
            pltpu.VMEM((C, ACC_W), jnp.float32),
            pltpu.VMEM((2, SG, 2, C), jnp.int32),
            pltpu.VMEM((C,), jnp.int32),
            pltpu.VMEM((C,), jnp.int32),
            pltpu.VMEM_SHARED((acc_n, ACC_W), jnp.float32),
            pltpu.SemaphoreType.DMA,
            pltpu.SemaphoreType.DMA,
            pltpu.SemaphoreType.DMA,
            pltpu.SemaphoreType.DMA,
            pltpu.SemaphoreType.DMA,
        ],
    )


def _combine_body(x_ref, p0_ref, p1_ref, o_ref):
    x = x_ref[...]
    wv = p0_ref[0, :, :D] + p1_ref[0, :, :D]
    z = p0_ref[0, :, D:D + H] + p1_ref[0, :, D:D + H]
    r = 1.0 / z
    for h in range(H):
        sl = slice(h * DH, (h + 1) * DH)
        o_ref[:, sl] = x[:, sl] + wv[:, sl] * r[:, h:h + 1]


@functools.lru_cache(maxsize=None)
def _combine_call(n, blk, acc_n):
    grid = n // blk
    return pl.pallas_call(
        _combine_body,
        grid=(grid,),
        in_specs=[
            pl.BlockSpec((blk, D), lambda i: (i, 0)),
            pl.BlockSpec((1, blk, ACC_W), lambda i: (0, i, 0)),
            pl.BlockSpec((1, blk, ACC_W), lambda i: (1, i, 0)),
        ],
        out_specs=pl.BlockSpec((blk, D), lambda i: (i, 0)),
        out_shape=jax.ShapeDtypeStruct((n, D), jnp.float32),
    )


def kernel(x, edge_index, Wq, bq, Wk, bk, Wv, bv):
    n = x.shape[0]
    e = edge_index.shape[1]

    acc_n = -(-(n + 1) // (NS * C)) * (NS * C)
    w_cat = jnp.concatenate([Wq, Wk, Wv], axis=1)
    b_cat = jnp.concatenate([bq, bk, bv]).reshape(1, 3 * D)
    x_pad = jnp.pad(x, ((0, acc_n - n), (0, 0)))
    q, kv = _qkv_call(acc_n, 2048)(x_pad, w_cat, b_cat)

    nch = -(-e // (NW * C))
    nch = -(-nch // SG) * SG
    e_pad = nch * C * NW
    npad = e_pad - e
    src = jnp.concatenate(
        [edge_index[0], jnp.zeros((npad,), jnp.int32)])
    dst = jnp.concatenate(
        [edge_index[1], n + (jnp.arange(npad, dtype=jnp.int32)
                             % (acc_n - n))])
    e2d = jnp.stack([src.reshape(-1, C), dst.reshape(-1, C)], axis=1)

    partials = _sc_edge_call(nch, acc_n)(q, kv, e2d)

    return _combine_call(n, 2000, acc_n)(x, partials, partials)
